# baseline (speedup 1.0000x reference)
import functools

import jax
import jax.numpy as jnp
from jax import lax
from jax.experimental import pallas as pl
from jax.experimental.pallas import tpu as pltpu
from jax.experimental.pallas import tpu_sc as plsc

_NUM_USERS = 25000
_N_NODES = 50000
_N_EDGES = 800000
_D = 64
_LAYERS = 3
_GAMMA = 0.4
_C1 = (1.0 - _GAMMA) / 3.0
_REG = 1e-4
_BATCH = 4096

_NC = 2
_NS = 16
_HALF_E = _N_EDGES // 2
_HALF_N = _N_NODES // 2
_K = 128
_CHUNKS = _HALF_E // _K
_WB = 1560
_WREM = _HALF_N - _NS * _WB
_ZN = _WB // _K
_ZREM = _WB - _ZN * _K

_M = 5
_ME = _M * _K
_BLKS = _CHUNKS // _M
_BPTS = _BLKS // _NS
_BREM = _BLKS % _NS

_NB = 3 * _BATCH
_BPT = _NB // (_NC * _NS)
_BK = 128
_BC = _BPT // _BK


def _mesh():
    return plsc.VectorSubcoreMesh(
        core_axis_name="c", subcore_axis_name="s",
        num_cores=_NC, num_subcores=_NS)


def _normalize_body(x_ref, o_ref):
    x = x_ref[...]
    ss = jnp.sum(x * x, axis=1, keepdims=True)
    xn = x * (1.0 / (jnp.sqrt(ss) + 1e-12))
    parts = []
    for g in range(_D // 32):
        a = xn[:, g * 32:g * 32 + 16].astype(jnp.bfloat16)
        b = xn[:, g * 32 + 16:g * 32 + 32].astype(jnp.bfloat16)
        a32 = lax.bitcast_convert_type(a, jnp.uint16).astype(jnp.uint32)
        b32 = lax.bitcast_convert_type(b, jnp.uint16).astype(jnp.uint32)
        parts.append(a32 | (b32 << 16))
    o_ref[...] = jnp.concatenate(parts, axis=1)


def _normalize(emb):
    rows = 2000
    return pl.pallas_call(
        _normalize_body,
        grid=(_N_NODES // rows,),
        in_specs=[pl.BlockSpec((rows, _D), lambda i: (i, 0))],
        out_specs=pl.BlockSpec((rows, _D // 2), lambda i: (i, 0)),
        out_shape=jax.ShapeDtypeStruct((_N_NODES, _D // 2), jnp.uint32),
    )(emb)


def _spmm_body(pk_h, row_h, col_h, val_h, out_h,
               acc, colm_v, rowm_v, valm_v,
               col0_v, col1_v, row0_v, row1_v,
               gat0_v, gat1_v, sca0_v, sca1_v,
               semg0, semg1, sems0, sems1):
    c = lax.axis_index("c")
    s = lax.axis_index("s")
    col_b = (col0_v, col1_v)
    row_b = (row0_v, row1_v)
    gat_b = (gat0_v, gat1_v)
    sca_b = (sca0_v, sca1_v)
    semg = (semg0, semg1)
    sems = (sems0, sems1)

    def zrow(i, carry):
        for d in range(_D // 16):
            sca0_v[i, pl.ds(d * 16, 16)] = jnp.zeros((16,), jnp.float32)
        return carry
    lax.fori_loop(0, _K, zrow, 0)
    r0 = s * _WB

    def zcopy(jz, carry):
        pltpu.sync_copy(sca0_v, acc.at[pl.ds(r0 + jz * _K, _K)])
        return carry
    lax.fori_loop(0, _ZN, zcopy, 0)
    pltpu.sync_copy(sca0_v.at[pl.ds(0, _ZREM)],
                    acc.at[pl.ds(r0 + _ZN * _K, _ZREM)])

    @pl.when(s == 0)
    def _():
        pltpu.sync_copy(sca0_v.at[pl.ds(0, _WREM)],
                        acc.at[pl.ds(_NS * _WB, _WREM)])
    plsc.subcore_barrier()

    base_n = c * _HALF_N
    start = s * _BPTS + jnp.minimum(s, _BREM)
    cnt = _BPTS + jnp.where(s < _BREM, 1, 0)

    def build_col(j, b):
        @plsc.parallel_loop(0, _K // 16, unroll=4)
        def _(g):
            sl = pl.ds(g * 16, 16)
            col_b[b][sl] = colm_v[pl.ds(j * _K + g * 16, 16)]

    def block(bi, carry):
        e0 = c * _HALF_E + bi * _ME
        pltpu.sync_copy(col_h.at[pl.ds(e0, _ME)], colm_v)
        pltpu.sync_copy(row_h.at[pl.ds(e0, _ME)], rowm_v)
        pltpu.sync_copy(val_h.at[pl.ds(e0, _ME)], valm_v)
        build_col(0, 0)
        pltpu.async_copy(pk_h.at[col0_v], gat0_v, semg0)

        def chunk(j, carry2):
            def step(b):
                nb = 1 - b

                @pl.when(j < _M - 1)
                def _():
                    @pl.when(j >= 1)
                    def _():
                        pltpu.make_async_copy(
                            sca_b[nb], acc.at[row_b[nb]], sems[nb]).wait()
                    build_col(jnp.int32(j) + 1, nb)
                    pltpu.async_copy(
                        pk_h.at[col_b[nb]], gat_b[nb], semg[nb])
                pltpu.make_async_copy(
                    pk_h.at[col_b[b]], gat_b[b], semg[b]).wait()

                @plsc.parallel_loop(0, _K // 16, unroll=4)
                def _(g):
                    sl = pl.ds(g * 16, 16)
                    row_b[b][sl] = rowm_v[pl.ds(j * _K + g * 16, 16)] - base_n

                @plsc.parallel_loop(0, _K, unroll=8)
                def _(k):
                    sc = plsc.load_gather(
                        valm_v, [jnp.full((16,), j * _K + k, jnp.int32)])
                    for g in range(_D // 32):
                        w = gat_b[b][k, pl.ds(g * 16, 16)]
                        lo = plsc.bitcast(w << 16, jnp.float32)
                        hi = plsc.bitcast(w & jnp.uint32(0xFFFF0000),
                                          jnp.float32)
                        sca_b[b][k, pl.ds(g * 32, 16)] = lo * sc
                        sca_b[b][k, pl.ds(g * 32 + 16, 16)] = hi * sc
                pltpu.make_async_copy(
                    sca_b[b], acc.at[row_b[b]], sems[b]).start(add=True)

            @pl.when(j % 2 == 0)
            def _():
                step(0)

            @pl.when(j % 2 == 1)
            def _():
                step(1)
            return carry2
        lax.fori_loop(0, _M, chunk, 0)
        pltpu.make_async_copy(sca0_v, acc.at[row0_v], sems0).wait()
        pltpu.make_async_copy(sca1_v, acc.at[row1_v], sems1).wait()
        return carry
    lax.fori_loop(start, start + cnt, block, 0)
    plsc.subcore_barrier()

    pltpu.sync_copy(acc.at[pl.ds(r0, _WB)],
                    out_h.at[pl.ds(base_n + r0, _WB)])

    @pl.when(s == 0)
    def _():
        pltpu.sync_copy(acc.at[pl.ds(_NS * _WB, _WREM)],
                        out_h.at[pl.ds(base_n + _NS * _WB, _WREM)])


def _spmm(pk, rows, cols, vals):
    f = pl.kernel(
        _spmm_body,
        out_type=jax.ShapeDtypeStruct((_N_NODES, _D), jnp.float32),
        mesh=_mesh(),
        scratch_types=[
            pltpu.VMEM_SHARED((_HALF_N, _D), jnp.float32),
            pltpu.VMEM((_ME,), jnp.int32),
            pltpu.VMEM((_ME,), jnp.int32),
            pltpu.VMEM((_ME,), jnp.float32),
            pltpu.VMEM((_K,), jnp.int32),
            pltpu.VMEM((_K,), jnp.int32),
            pltpu.VMEM((_K,), jnp.int32),
            pltpu.VMEM((_K,), jnp.int32),
            pltpu.VMEM((_K, _D // 2), jnp.uint32),
            pltpu.VMEM((_K, _D // 2), jnp.uint32),
            pltpu.VMEM((_K, _D), jnp.float32),
            pltpu.VMEM((_K, _D), jnp.float32),
            pltpu.SemaphoreType.DMA,
            pltpu.SemaphoreType.DMA,
            pltpu.SemaphoreType.DMA,
            pltpu.SemaphoreType.DMA,
        ],
        compiler_params=pltpu.CompilerParams(
            needs_layout_passes=False, use_tc_tiling_on_sc=False),
    )
    return f(pk, rows, cols, vals)


def _combine_body(e0_h, e1_h, e2_h, e3_h, idx_h, fin_h, ego_h,
                  idx_v, g0, g1, g2, g3, fin_v, sem):
    c = lax.axis_index("c")
    s = lax.axis_index("s")
    wid = s * _NC + c

    def chunk(ci, carry):
        b0 = wid * _BPT + ci * _BK
        pltpu.sync_copy(idx_h.at[pl.ds(b0, _BK)], idx_v)
        pltpu.async_copy(e0_h.at[idx_v], g0, sem).wait()
        pltpu.async_copy(e1_h.at[idx_v], g1, sem).wait()
        pltpu.async_copy(e2_h.at[idx_v], g2, sem).wait()
        pltpu.async_copy(e3_h.at[idx_v], g3, sem).wait()

        def comb(k, carry2):
            for d in range(_D // 16):
                sl = pl.ds(d * 16, 16)
                fin_v[k, sl] = (_GAMMA * g0[k, sl]
                                + _C1 * (g1[k, sl] + g2[k, sl] + g3[k, sl]))
            return carry2
        lax.fori_loop(0, _BK, comb, 0)
        pltpu.sync_copy(fin_v, fin_h.at[pl.ds(b0, _BK)])
        pltpu.sync_copy(g0, ego_h.at[pl.ds(b0, _BK)])
        return carry
    lax.fori_loop(0, _BC, chunk, 0)


def _gather_combine(e0, e1, e2, e3, idx):
    f = pl.kernel(
        _combine_body,
        out_type=(jax.ShapeDtypeStruct((_NB, _D), jnp.float32),
                  jax.ShapeDtypeStruct((_NB, _D), jnp.float32)),
        mesh=_mesh(),
        scratch_types=[
            pltpu.VMEM((_BK,), jnp.int32),
            pltpu.VMEM((_BK, _D), jnp.float32),
            pltpu.VMEM((_BK, _D), jnp.float32),
            pltpu.VMEM((_BK, _D), jnp.float32),
            pltpu.VMEM((_BK, _D), jnp.float32),
            pltpu.VMEM((_BK, _D), jnp.float32),
            pltpu.SemaphoreType.DMA,
        ],
        compiler_params=pltpu.CompilerParams(
            needs_layout_passes=False, use_tc_tiling_on_sc=False),
    )
    return f(e0, e1, e2, e3, idx)


def _loss_body(fin_ref, ego_ref, bpr_ref, reg_ref):
    f = fin_ref[...]
    u = f[0:_BATCH]
    p = f[_BATCH:2 * _BATCH]
    n = f[2 * _BATCH:3 * _BATCH]
    pos = jnp.sum(u * p, axis=1)
    neg = jnp.sum(u * n, axis=1)
    bpr = jnp.mean(jax.nn.softplus(neg - pos))
    e = ego_ref[...]
    reg = (0.5 * _REG / _BATCH) * jnp.sum(e * e)
    bpr_ref[...] = jnp.reshape(bpr, (1, 1))
    reg_ref[...] = jnp.reshape(reg, (1, 1))


def _loss(fin, ego):
    return pl.pallas_call(
        _loss_body,
        out_shape=(jax.ShapeDtypeStruct((1, 1), jnp.float32),
                   jax.ShapeDtypeStruct((1, 1), jnp.float32)),
    )(fin, ego)


def kernel(user, positive, negative, user_table, item_table,
           graph_row, graph_col, graph_val):
    emb = jnp.concatenate([user_table, item_table], axis=0)
    embs = [emb]
    for _ in range(_LAYERS):
        emb_n = _normalize(embs[-1])
        embs.append(_spmm(emb_n, graph_row, graph_col, graph_val))
    idx = jnp.concatenate(
        [user, positive + _NUM_USERS, negative + _NUM_USERS])
    fin, ego = _gather_combine(embs[0], embs[1], embs[2], embs[3], idx)
    bpr, reg = _loss(fin, ego)
    return (bpr[0, 0], reg[0, 0])


# speedup vs baseline: 1.2441x; 1.2441x over previous
import functools

import jax
import jax.numpy as jnp
from jax import lax
from jax.experimental import pallas as pl
from jax.experimental.pallas import tpu as pltpu
from jax.experimental.pallas import tpu_sc as plsc

_NUM_USERS = 25000
_N_NODES = 50000
_N_EDGES = 800000
_D = 64
_LAYERS = 3
_GAMMA = 0.4
_C1 = (1.0 - _GAMMA) / 3.0
_REG = 1e-4
_BATCH = 4096

_NC = 2
_NS = 16
_HALF_E = _N_EDGES // 2
_HALF_N = _N_NODES // 2
_K = 128
_CHUNKS = _HALF_E // _K
_WB = 1560
_WREM = _HALF_N - _NS * _WB
_ZN = _WB // _K
_ZREM = _WB - _ZN * _K

_M = 25
_ME = _M * _K
_BLKS = _CHUNKS // _M
_BPTS = _BLKS // _NS
_BREM = _BLKS % _NS

_NB = 3 * _BATCH
_BPT = _NB // (_NC * _NS)
_BK = 128
_BC = _BPT // _BK


def _mesh():
    return plsc.VectorSubcoreMesh(
        core_axis_name="c", subcore_axis_name="s",
        num_cores=_NC, num_subcores=_NS)


def _normalize_body(x_ref, o_ref):
    x = x_ref[...]
    ss = jnp.sum(x * x, axis=1, keepdims=True)
    xn = x * (1.0 / (jnp.sqrt(ss) + 1e-12))
    parts = []
    for g in range(_D // 32):
        a = xn[:, g * 32:g * 32 + 16].astype(jnp.bfloat16)
        b = xn[:, g * 32 + 16:g * 32 + 32].astype(jnp.bfloat16)
        a32 = lax.bitcast_convert_type(a, jnp.uint16).astype(jnp.uint32)
        b32 = lax.bitcast_convert_type(b, jnp.uint16).astype(jnp.uint32)
        parts.append(a32 | (b32 << 16))
    o_ref[...] = jnp.concatenate(parts, axis=1)


def _normalize(emb):
    rows = 2000
    return pl.pallas_call(
        _normalize_body,
        grid=(_N_NODES // rows,),
        in_specs=[pl.BlockSpec((rows, _D), lambda i: (i, 0))],
        out_specs=pl.BlockSpec((rows, _D // 2), lambda i: (i, 0)),
        out_shape=jax.ShapeDtypeStruct((_N_NODES, _D // 2), jnp.uint32),
    )(emb)


def _spmm_body(pk_h, row_h, col_h, val_h, out_h,
               acc, colm_v, rowm_v, valm_v,
               col0_v, col1_v, row0_v, row1_v,
               gat0_v, gat1_v, sca_v,
               semg0, semg1):
    c = lax.axis_index("c")
    s = lax.axis_index("s")
    col_b = (col0_v, col1_v)
    row_b = (row0_v, row1_v)
    gat_b = (gat0_v, gat1_v)
    semg = (semg0, semg1)

    def zrow(i, carry):
        for d in range(_D // 16):
            sca_v[i, pl.ds(d * 16, 16)] = jnp.zeros((16,), jnp.float32)
        return carry
    lax.fori_loop(0, _K, zrow, 0)
    r0 = s * _WB

    def zcopy(jz, carry):
        pltpu.sync_copy(sca_v, acc.at[pl.ds(r0 + jz * _K, _K)])
        return carry
    lax.fori_loop(0, _ZN, zcopy, 0)
    pltpu.sync_copy(sca_v.at[pl.ds(0, _ZREM)],
                    acc.at[pl.ds(r0 + _ZN * _K, _ZREM)])

    @pl.when(s == 0)
    def _():
        pltpu.sync_copy(sca_v.at[pl.ds(0, _WREM)],
                        acc.at[pl.ds(_NS * _WB, _WREM)])
    plsc.subcore_barrier()

    base_n = c * _HALF_N
    start = s * _BPTS + jnp.minimum(s, _BREM)
    cnt = _BPTS + jnp.where(s < _BREM, 1, 0)

    def build_col(j, b):
        @plsc.parallel_loop(0, _K // 16, unroll=4)
        def _(g):
            sl = pl.ds(g * 16, 16)
            col_b[b][sl] = colm_v[pl.ds(j * _K + g * 16, 16)]

    def block(bi, carry):
        e0 = c * _HALF_E + bi * _ME
        pltpu.sync_copy(col_h.at[pl.ds(e0, _ME)], colm_v)
        pltpu.sync_copy(row_h.at[pl.ds(e0, _ME)], rowm_v)
        pltpu.sync_copy(val_h.at[pl.ds(e0, _ME)], valm_v)
        build_col(0, 0)
        pltpu.async_copy(pk_h.at[col0_v], gat0_v, semg0)

        def chunk(j, carry2):
            def step(b):
                nb = 1 - b

                @pl.when(j < _M - 1)
                def _():
                    build_col(jnp.int32(j) + 1, nb)
                    pltpu.async_copy(
                        pk_h.at[col_b[nb]], gat_b[nb], semg[nb])
                pltpu.make_async_copy(
                    pk_h.at[col_b[b]], gat_b[b], semg[b]).wait()

                @plsc.parallel_loop(0, _K // 16, unroll=4)
                def _(g):
                    sl = pl.ds(g * 16, 16)
                    row_b[b][sl] = rowm_v[pl.ds(j * _K + g * 16, 16)] - base_n

                @plsc.parallel_loop(0, _K, unroll=8)
                def _(k):
                    sc = plsc.load_gather(
                        valm_v, [jnp.full((16,), j * _K + k, jnp.int32)])
                    for g in range(_D // 32):
                        w = gat_b[b][k, pl.ds(g * 16, 16)]
                        lo = plsc.bitcast(w << 16, jnp.float32)
                        hi = plsc.bitcast(w & jnp.uint32(0xFFFF0000),
                                          jnp.float32)
                        sca_v[k, pl.ds(g * 32, 16)] = lo * sc
                        sca_v[k, pl.ds(g * 32 + 16, 16)] = hi * sc
                pltpu.sync_copy(sca_v, acc.at[row_b[b]], add=True)

            @pl.when(j % 2 == 0)
            def _():
                step(0)

            @pl.when(j % 2 == 1)
            def _():
                step(1)
            return carry2
        lax.fori_loop(0, _M, chunk, 0)
        return carry
    lax.fori_loop(start, start + cnt, block, 0)
    plsc.subcore_barrier()

    pltpu.sync_copy(acc.at[pl.ds(r0, _WB)],
                    out_h.at[pl.ds(base_n + r0, _WB)])

    @pl.when(s == 0)
    def _():
        pltpu.sync_copy(acc.at[pl.ds(_NS * _WB, _WREM)],
                        out_h.at[pl.ds(base_n + _NS * _WB, _WREM)])


def _spmm(pk, rows, cols, vals):
    f = pl.kernel(
        _spmm_body,
        out_type=jax.ShapeDtypeStruct((_N_NODES, _D), jnp.float32),
        mesh=_mesh(),
        scratch_types=[
            pltpu.VMEM_SHARED((_HALF_N, _D), jnp.float32),
            pltpu.VMEM((_ME,), jnp.int32),
            pltpu.VMEM((_ME,), jnp.int32),
            pltpu.VMEM((_ME,), jnp.float32),
            pltpu.VMEM((_K,), jnp.int32),
            pltpu.VMEM((_K,), jnp.int32),
            pltpu.VMEM((_K,), jnp.int32),
            pltpu.VMEM((_K,), jnp.int32),
            pltpu.VMEM((_K, _D // 2), jnp.uint32),
            pltpu.VMEM((_K, _D // 2), jnp.uint32),
            pltpu.VMEM((_K, _D), jnp.float32),
            pltpu.SemaphoreType.DMA,
            pltpu.SemaphoreType.DMA,
        ],
        compiler_params=pltpu.CompilerParams(
            needs_layout_passes=False, use_tc_tiling_on_sc=False),
    )
    return f(pk, rows, cols, vals)


def _combine_body(e0_h, e1_h, e2_h, e3_h, idx_h, fin_h, ego_h,
                  idx_v, g0, g1, g2, g3, fin_v, sem):
    c = lax.axis_index("c")
    s = lax.axis_index("s")
    wid = s * _NC + c

    def chunk(ci, carry):
        b0 = wid * _BPT + ci * _BK
        pltpu.sync_copy(idx_h.at[pl.ds(b0, _BK)], idx_v)
        pltpu.async_copy(e0_h.at[idx_v], g0, sem).wait()
        pltpu.async_copy(e1_h.at[idx_v], g1, sem).wait()
        pltpu.async_copy(e2_h.at[idx_v], g2, sem).wait()
        pltpu.async_copy(e3_h.at[idx_v], g3, sem).wait()

        def comb(k, carry2):
            for d in range(_D // 16):
                sl = pl.ds(d * 16, 16)
                fin_v[k, sl] = (_GAMMA * g0[k, sl]
                                + _C1 * (g1[k, sl] + g2[k, sl] + g3[k, sl]))
            return carry2
        lax.fori_loop(0, _BK, comb, 0)
        pltpu.sync_copy(fin_v, fin_h.at[pl.ds(b0, _BK)])
        pltpu.sync_copy(g0, ego_h.at[pl.ds(b0, _BK)])
        return carry
    lax.fori_loop(0, _BC, chunk, 0)


def _gather_combine(e0, e1, e2, e3, idx):
    f = pl.kernel(
        _combine_body,
        out_type=(jax.ShapeDtypeStruct((_NB, _D), jnp.float32),
                  jax.ShapeDtypeStruct((_NB, _D), jnp.float32)),
        mesh=_mesh(),
        scratch_types=[
            pltpu.VMEM((_BK,), jnp.int32),
            pltpu.VMEM((_BK, _D), jnp.float32),
            pltpu.VMEM((_BK, _D), jnp.float32),
            pltpu.VMEM((_BK, _D), jnp.float32),
            pltpu.VMEM((_BK, _D), jnp.float32),
            pltpu.VMEM((_BK, _D), jnp.float32),
            pltpu.SemaphoreType.DMA,
        ],
        compiler_params=pltpu.CompilerParams(
            needs_layout_passes=False, use_tc_tiling_on_sc=False),
    )
    return f(e0, e1, e2, e3, idx)


def _loss_body(fin_ref, ego_ref, bpr_ref, reg_ref):
    f = fin_ref[...]
    u = f[0:_BATCH]
    p = f[_BATCH:2 * _BATCH]
    n = f[2 * _BATCH:3 * _BATCH]
    pos = jnp.sum(u * p, axis=1)
    neg = jnp.sum(u * n, axis=1)
    bpr = jnp.mean(jax.nn.softplus(neg - pos))
    e = ego_ref[...]
    reg = (0.5 * _REG / _BATCH) * jnp.sum(e * e)
    bpr_ref[...] = jnp.reshape(bpr, (1, 1))
    reg_ref[...] = jnp.reshape(reg, (1, 1))


def _loss(fin, ego):
    return pl.pallas_call(
        _loss_body,
        out_shape=(jax.ShapeDtypeStruct((1, 1), jnp.float32),
                   jax.ShapeDtypeStruct((1, 1), jnp.float32)),
    )(fin, ego)


def kernel(user, positive, negative, user_table, item_table,
           graph_row, graph_col, graph_val):
    emb = jnp.concatenate([user_table, item_table], axis=0)
    embs = [emb]
    for _ in range(_LAYERS):
        emb_n = _normalize(embs[-1])
        embs.append(_spmm(emb_n, graph_row, graph_col, graph_val))
    idx = jnp.concatenate(
        [user, positive + _NUM_USERS, negative + _NUM_USERS])
    fin, ego = _gather_combine(embs[0], embs[1], embs[2], embs[3], idx)
    bpr, reg = _loss(fin, ego)
    return (bpr[0, 0], reg[0, 0])
